# baseline (speedup 1.0000x reference)
import functools

import jax
import jax.numpy as jnp
from jax import lax
from jax.experimental import pallas as pl
from jax.experimental.pallas import tpu as pltpu
from jax.experimental.pallas import tpu_sc as plsc

E = 64
D = 1024
F = 1024
S = 16384
C = 320
EC = E * C
NBLK = E + 2

NB = 8
B = S // NB
BC = 256

NW = 32
TPW = S // NW
CHUNK = 64
NCHUNK = TPW // CHUNK
GCHUNK = 64
NGCHUNK = TPW // GCHUNK


def _router_body(x_ref, wg_ref, slot_ref, xs_ref, counts_ref, lt_ref):
    step = pl.program_id(0)

    @pl.when(step == 0)
    def _init():
        counts_ref[...] = jnp.zeros_like(counts_ref)
        row = lax.broadcasted_iota(jnp.int32, (BC, BC), 0)
        col = lax.broadcasted_iota(jnp.int32, (BC, BC), 1)
        lt_ref[...] = (col < row).astype(jnp.bfloat16)

    x = x_ref[...]
    wg = wg_ref[...]
    logits = jnp.dot(x.astype(jnp.bfloat16), wg.astype(jnp.bfloat16),
                     preferred_element_type=jnp.float32)
    m = jnp.max(logits, axis=1, keepdims=True)
    ssum = jnp.sum(jnp.exp(logits - m), axis=1)
    gate = 1.0 / ssum
    eidx = lax.broadcasted_iota(jnp.int32, (B, E), 1)
    is_max = logits == m
    expert = jnp.min(jnp.where(is_max, eidx, E), axis=1)
    onehot = (eidx == expert[:, None])
    onehot_f = onehot.astype(jnp.float32)
    onehot_bf = onehot.astype(jnp.bfloat16)
    lt = lt_ref[...]
    base = counts_ref[0:1, 0:E]
    chunks = []
    for c in range(B // BC):
        oh_c = onehot_bf[c * BC : (c + 1) * BC]
        pos_c = jnp.dot(lt, oh_c, preferred_element_type=jnp.float32)
        chunks.append(pos_c + base)
        base = base + jnp.sum(onehot_f[c * BC : (c + 1) * BC],
                              axis=0, keepdims=True)
    pos_excl = jnp.concatenate(chunks, axis=0)
    posf = jnp.sum(pos_excl * onehot_f, axis=1)
    pos = posf.astype(jnp.int32)
    counts_ref[0:1, 0:E] = base
    keep = pos < C
    slot = jnp.where(keep, expert * C + pos, EC)
    gate = jnp.where(keep, gate, 0.0)
    slot_ref[0, 0, :] = slot
    xb = (x * gate[:, None]).astype(jnp.bfloat16)
    lo = lax.bitcast_convert_type(xb[:, : D // 2], jnp.uint16).astype(jnp.uint32)
    hi = lax.bitcast_convert_type(xb[:, D // 2 :], jnp.uint16).astype(jnp.uint32)
    xs_ref[...] = lax.bitcast_convert_type(lo | (hi << 16), jnp.int32)


def _router(inp, Wg):
    return pl.pallas_call(
        _router_body,
        grid=(NB,),
        in_specs=[
            pl.BlockSpec((B, D), lambda i: (i, 0)),
            pl.BlockSpec((D, E), lambda i: (0, 0)),
        ],
        out_specs=[
            pl.BlockSpec((1, 1, B), lambda i: (i, 0, 0)),
            pl.BlockSpec((B, D // 2), lambda i: (i, 0)),
        ],
        out_shape=[
            jax.ShapeDtypeStruct((NB, 1, B), jnp.int32),
            jax.ShapeDtypeStruct((S, D // 2), jnp.int32),
        ],
        scratch_shapes=[
            pltpu.VMEM((1, E), jnp.float32),
            pltpu.VMEM((BC, BC), jnp.bfloat16),
        ],
        compiler_params=pltpu.CompilerParams(
            dimension_semantics=("arbitrary",)),
    )(inp, Wg)


@functools.cache
def _make_dispatch():
    mesh = plsc.VectorSubcoreMesh(core_axis_name="c", subcore_axis_name="s")

    @functools.partial(
        pl.kernel,
        out_type=jax.ShapeDtypeStruct((NBLK * C, D // 2), jnp.int32),
        mesh=mesh,
        scratch_types=[
            pltpu.VMEM((2, CHUNK), jnp.int32),
            pltpu.VMEM((2, CHUNK, D // 2), jnp.int32),
            pltpu.SemaphoreType.DMA,
            pltpu.SemaphoreType.DMA,
            pltpu.SemaphoreType.DMA,
            pltpu.SemaphoreType.DMA,
        ],
    )
    def _dispatch(xs_hbm, slot_hbm, disp_hbm, idx_v, rows_v,
                  sem_in0, sem_in1, sem_sc0, sem_sc1):
        wid = lax.axis_index("s") * 2 + lax.axis_index("c")
        base = wid * TPW
        sem_in = (sem_in0, sem_in1)
        sem_sc = (sem_sc0, sem_sc1)

        scat = [None, None]
        for j in range(NCHUNK):
            b = j & 1
            if scat[b] is not None:
                scat[b].wait()
            off = base + j * CHUNK
            ci = pltpu.async_copy(slot_hbm.at[pl.ds(off, CHUNK)],
                                  idx_v.at[b], sem_in[b])
            cr = pltpu.async_copy(xs_hbm.at[pl.ds(off, CHUNK)],
                                  rows_v.at[b], sem_sc[b])
            ci.wait()
            cr.wait()
            scat[b] = pltpu.async_copy(rows_v.at[b], disp_hbm.at[idx_v.at[b]],
                                       sem_in[b])
        for s in scat:
            if s is not None:
                s.wait()

    return _dispatch


def _unpack_bf16_pairs(p):
    lo = lax.bitcast_convert_type(
        (p & 0xFFFF).astype(jnp.uint16), jnp.bfloat16)
    hi = lax.bitcast_convert_type(
        lax.shift_right_logical(p, 16).astype(jnp.uint16), jnp.bfloat16)
    return jnp.concatenate([lo, hi], axis=1)


def _pack_bf16_pairs(xb):
    lo = lax.bitcast_convert_type(xb[:, : D // 2], jnp.uint16).astype(jnp.uint32)
    hi = lax.bitcast_convert_type(xb[:, D // 2 :], jnp.uint16).astype(jnp.uint32)
    return lax.bitcast_convert_type(lo | (hi << 16), jnp.int32)


def _ffn_body(x_ref, w1_ref, w2_ref, o_ref):
    g = pl.program_id(0)
    for k in range(2):
        x = _unpack_bf16_pairs(x_ref[k])
        w1 = w1_ref[k].astype(jnp.bfloat16)
        h = jnp.dot(x, w1, preferred_element_type=jnp.float32)
        h = jnp.maximum(h, 0.0).astype(jnp.bfloat16)
        w2 = w2_ref[k].astype(jnp.bfloat16)
        y = jnp.dot(h, w2, preferred_element_type=jnp.float32)
        packed = _pack_bf16_pairs(y.astype(jnp.bfloat16))
        o_ref[k] = jnp.where(g == NBLK // 2 - 1, 0, packed)


def _ffn(disp, W1, W2):
    return pl.pallas_call(
        _ffn_body,
        grid=(NBLK // 2,),
        in_specs=[
            pl.BlockSpec((2, C, D // 2), lambda g: (g, 0, 0)),
            pl.BlockSpec((2, D, F), lambda g: (jnp.minimum(g, E // 2 - 1), 0, 0)),
            pl.BlockSpec((2, F, D), lambda g: (jnp.minimum(g, E // 2 - 1), 0, 0)),
        ],
        out_specs=pl.BlockSpec((2, C, D // 2), lambda g: (g, 0, 0)),
        out_shape=jax.ShapeDtypeStruct((NBLK, C, D // 2), jnp.int32),
        compiler_params=pltpu.CompilerParams(
            dimension_semantics=("parallel",)),
    )(disp, W1, W2)


@functools.cache
def _make_combine():
    mesh = plsc.VectorSubcoreMesh(core_axis_name="c", subcore_axis_name="s")

    @functools.partial(
        pl.kernel,
        out_type=jax.ShapeDtypeStruct((S, D // 2), jnp.int32),
        mesh=mesh,
        scratch_types=[
            pltpu.VMEM((2, GCHUNK), jnp.int32),
            pltpu.VMEM((2, GCHUNK, D // 2), jnp.int32),
            pltpu.SemaphoreType.DMA,
            pltpu.SemaphoreType.DMA,
            pltpu.SemaphoreType.DMA,
            pltpu.SemaphoreType.DMA,
        ],
    )
    def _combine(eout_hbm, slot_hbm, out_hbm, idx_v, rows_v,
                 sem_g0, sem_g1, sem_w0, sem_w1):
        wid = lax.axis_index("s") * 2 + lax.axis_index("c")
        base = wid * TPW
        sem_g = (sem_g0, sem_g1)
        sem_w = (sem_w0, sem_w1)

        gath = [None, None]
        wr = [None, None]
        for j in range(NGCHUNK):
            b = j & 1
            if wr[b] is not None:
                wr[b].wait()
            off = base + j * GCHUNK
            pltpu.sync_copy(slot_hbm.at[pl.ds(off, GCHUNK)], idx_v.at[b])
            gath[b] = pltpu.async_copy(eout_hbm.at[idx_v.at[b]],
                                       rows_v.at[b], sem_g[b])
            o = b ^ 1
            if gath[o] is not None:
                gath[o].wait()
                po = base + (j - 1) * GCHUNK
                wr[o] = pltpu.async_copy(rows_v.at[o],
                                         out_hbm.at[pl.ds(po, GCHUNK)],
                                         sem_w[o])
                gath[o] = None
        b = (NGCHUNK - 1) & 1
        gath[b].wait()
        pltpu.async_copy(rows_v.at[b],
                         out_hbm.at[pl.ds(base + (NGCHUNK - 1) * GCHUNK, GCHUNK)],
                         sem_w[b]).wait()
        if wr[b ^ 1] is not None:
            wr[b ^ 1].wait()

    return _combine


def kernel(inp, Wg, W1, W2):
    slot3, xs = _router(inp, Wg)
    slot = slot3.reshape(S)
    disp = _make_dispatch()(xs, slot)
    eout = _ffn(disp.reshape(NBLK, C, D // 2), W1, W2)
    pk = _make_combine()(eout.reshape(NBLK * C, D // 2), slot)
    lo = lax.bitcast_convert_type(
        (pk & 0xFFFF).astype(jnp.uint16), jnp.bfloat16).astype(jnp.float32)
    hi = lax.bitcast_convert_type(
        lax.shift_right_logical(pk, 16).astype(jnp.uint16),
        jnp.bfloat16).astype(jnp.float32)
    return jnp.concatenate([lo, hi], axis=1)


# speedup vs baseline: 1.0513x; 1.0513x over previous
import functools

import jax
import jax.numpy as jnp
from jax import lax
from jax.experimental import pallas as pl
from jax.experimental.pallas import tpu as pltpu
from jax.experimental.pallas import tpu_sc as plsc

E = 64
D = 1024
F = 1024
S = 16384
C = 320
EC = E * C
NBLK = E + 1

NB = 8
B = S // NB
BC = 256

NW = 32
TPW = S // NW
CHUNK = 64
NCHUNK = TPW // CHUNK
GCHUNK = 32
NGCHUNK = TPW // GCHUNK


def _router_body(x_ref, wg_ref, slot_ref, xs_ref, counts_ref, lt_ref):
    step = pl.program_id(0)

    @pl.when(step == 0)
    def _init():
        counts_ref[...] = jnp.zeros_like(counts_ref)
        row = lax.broadcasted_iota(jnp.int32, (BC, BC), 0)
        col = lax.broadcasted_iota(jnp.int32, (BC, BC), 1)
        lt_ref[...] = (col < row).astype(jnp.bfloat16)

    x = x_ref[...]
    wg = wg_ref[...]
    logits = jnp.dot(x.astype(jnp.bfloat16), wg.astype(jnp.bfloat16),
                     preferred_element_type=jnp.float32)
    m = jnp.max(logits, axis=1, keepdims=True)
    ssum = jnp.sum(jnp.exp(logits - m), axis=1)
    gate = 1.0 / ssum
    eidx = lax.broadcasted_iota(jnp.int32, (B, E), 1)
    is_max = logits == m
    expert = jnp.min(jnp.where(is_max, eidx, E), axis=1)
    onehot = (eidx == expert[:, None])
    onehot_f = onehot.astype(jnp.float32)
    onehot_bf = onehot.astype(jnp.bfloat16)
    lt = lt_ref[...]
    base = counts_ref[0:1, 0:E]
    chunks = []
    for c in range(B // BC):
        oh_c = onehot_bf[c * BC : (c + 1) * BC]
        pos_c = jnp.dot(lt, oh_c, preferred_element_type=jnp.float32)
        chunks.append(pos_c + base)
        base = base + jnp.sum(onehot_f[c * BC : (c + 1) * BC],
                              axis=0, keepdims=True)
    pos_excl = jnp.concatenate(chunks, axis=0)
    posf = jnp.sum(pos_excl * onehot_f, axis=1)
    pos = posf.astype(jnp.int32)
    counts_ref[0:1, 0:E] = base
    keep = pos < C
    slot = jnp.where(keep, expert * C + pos, EC)
    gate = jnp.where(keep, gate, 0.0)
    slot_ref[0, 0, :] = slot
    xb = (x * gate[:, None]).astype(jnp.bfloat16)
    lo = lax.bitcast_convert_type(xb[:, : D // 2], jnp.uint16).astype(jnp.uint32)
    hi = lax.bitcast_convert_type(xb[:, D // 2 :], jnp.uint16).astype(jnp.uint32)
    xs_ref[...] = lax.bitcast_convert_type(lo | (hi << 16), jnp.int32)


def _router(inp, Wg):
    return pl.pallas_call(
        _router_body,
        grid=(NB,),
        in_specs=[
            pl.BlockSpec((B, D), lambda i: (i, 0)),
            pl.BlockSpec((D, E), lambda i: (0, 0)),
        ],
        out_specs=[
            pl.BlockSpec((1, 1, B), lambda i: (i, 0, 0)),
            pl.BlockSpec((B, D // 2), lambda i: (i, 0)),
        ],
        out_shape=[
            jax.ShapeDtypeStruct((NB, 1, B), jnp.int32),
            jax.ShapeDtypeStruct((S, D // 2), jnp.int32),
        ],
        scratch_shapes=[
            pltpu.VMEM((1, E), jnp.float32),
            pltpu.VMEM((BC, BC), jnp.bfloat16),
        ],
        compiler_params=pltpu.CompilerParams(
            dimension_semantics=("arbitrary",)),
    )(inp, Wg)


@functools.cache
def _make_dispatch():
    mesh = plsc.VectorSubcoreMesh(core_axis_name="c", subcore_axis_name="s")

    @functools.partial(
        pl.kernel,
        out_type=jax.ShapeDtypeStruct((NBLK * C, D // 2), jnp.int32),
        mesh=mesh,
        scratch_types=[
            pltpu.VMEM((2, CHUNK), jnp.int32),
            pltpu.VMEM((2, CHUNK, D // 2), jnp.int32),
            pltpu.SemaphoreType.DMA,
            pltpu.SemaphoreType.DMA,
            pltpu.SemaphoreType.DMA,
            pltpu.SemaphoreType.DMA,
        ],
    )
    def _dispatch(xs_hbm, slot_hbm, disp_hbm, idx_v, rows_v,
                  sem_in0, sem_in1, sem_sc0, sem_sc1):
        wid = lax.axis_index("s") * 2 + lax.axis_index("c")
        base = wid * TPW
        sem_in = (sem_in0, sem_in1)
        sem_sc = (sem_sc0, sem_sc1)

        scat = [None, None]
        for j in range(NCHUNK):
            b = j & 1
            if scat[b] is not None:
                scat[b].wait()
            off = base + j * CHUNK
            ci = pltpu.async_copy(slot_hbm.at[pl.ds(off, CHUNK)],
                                  idx_v.at[b], sem_in[b])
            cr = pltpu.async_copy(xs_hbm.at[pl.ds(off, CHUNK)],
                                  rows_v.at[b], sem_sc[b])
            ci.wait()
            cr.wait()
            scat[b] = pltpu.async_copy(rows_v.at[b], disp_hbm.at[idx_v.at[b]],
                                       sem_in[b])
        for s in scat:
            if s is not None:
                s.wait()

    return _dispatch


def _unpack_bf16_pairs(p):
    lo = lax.bitcast_convert_type(
        (p & 0xFFFF).astype(jnp.uint16), jnp.bfloat16)
    hi = lax.bitcast_convert_type(
        lax.shift_right_logical(p, 16).astype(jnp.uint16), jnp.bfloat16)
    return jnp.concatenate([lo, hi], axis=1)


def _pack_bf16_pairs(xb):
    lo = lax.bitcast_convert_type(xb[:, : D // 2], jnp.uint16).astype(jnp.uint32)
    hi = lax.bitcast_convert_type(xb[:, D // 2 :], jnp.uint16).astype(jnp.uint32)
    return lax.bitcast_convert_type(lo | (hi << 16), jnp.int32)


def _ffn_body(x_ref, w1_ref, w2_ref, o_ref):
    e = pl.program_id(0)
    x = _unpack_bf16_pairs(x_ref[0])
    w1 = w1_ref[0].astype(jnp.bfloat16)
    h = jnp.dot(x, w1, preferred_element_type=jnp.float32)
    h = jnp.maximum(h, 0.0).astype(jnp.bfloat16)
    w2 = w2_ref[0].astype(jnp.bfloat16)
    y = jnp.dot(h, w2, preferred_element_type=jnp.float32)
    o_ref[0] = jnp.where(e == E, 0.0, y)


def _ffn(disp, W1, W2):
    return pl.pallas_call(
        _ffn_body,
        grid=(NBLK,),
        in_specs=[
            pl.BlockSpec((1, C, D // 2), lambda e: (e, 0, 0)),
            pl.BlockSpec((1, D, F), lambda e: (jnp.minimum(e, E - 1), 0, 0)),
            pl.BlockSpec((1, F, D), lambda e: (jnp.minimum(e, E - 1), 0, 0)),
        ],
        out_specs=pl.BlockSpec((1, C, D), lambda e: (e, 0, 0)),
        out_shape=jax.ShapeDtypeStruct((NBLK, C, D), jnp.float32),
        compiler_params=pltpu.CompilerParams(
            dimension_semantics=("parallel",)),
    )(disp, W1, W2)


@functools.cache
def _make_combine():
    mesh = plsc.VectorSubcoreMesh(core_axis_name="c", subcore_axis_name="s")

    @functools.partial(
        pl.kernel,
        out_type=jax.ShapeDtypeStruct((S, D), jnp.float32),
        mesh=mesh,
        scratch_types=[
            pltpu.VMEM((2, GCHUNK), jnp.int32),
            pltpu.VMEM((2, GCHUNK, D), jnp.float32),
            pltpu.SemaphoreType.DMA,
            pltpu.SemaphoreType.DMA,
            pltpu.SemaphoreType.DMA,
            pltpu.SemaphoreType.DMA,
        ],
    )
    def _combine(eout_hbm, slot_hbm, out_hbm, idx_v, rows_v,
                 sem_g0, sem_g1, sem_w0, sem_w1):
        wid = lax.axis_index("s") * 2 + lax.axis_index("c")
        base = wid * TPW
        sem_g = (sem_g0, sem_g1)
        sem_w = (sem_w0, sem_w1)

        gath = [None, None]
        wr = [None, None]
        for j in range(NGCHUNK):
            b = j & 1
            if wr[b] is not None:
                wr[b].wait()
            off = base + j * GCHUNK
            pltpu.sync_copy(slot_hbm.at[pl.ds(off, GCHUNK)], idx_v.at[b])
            gath[b] = pltpu.async_copy(eout_hbm.at[idx_v.at[b]],
                                       rows_v.at[b], sem_g[b])
            o = b ^ 1
            if gath[o] is not None:
                gath[o].wait()
                po = base + (j - 1) * GCHUNK
                wr[o] = pltpu.async_copy(rows_v.at[o],
                                         out_hbm.at[pl.ds(po, GCHUNK)],
                                         sem_w[o])
                gath[o] = None
        b = (NGCHUNK - 1) & 1
        gath[b].wait()
        pltpu.async_copy(rows_v.at[b],
                         out_hbm.at[pl.ds(base + (NGCHUNK - 1) * GCHUNK, GCHUNK)],
                         sem_w[b]).wait()
        if wr[b ^ 1] is not None:
            wr[b ^ 1].wait()

    return _combine


def kernel(inp, Wg, W1, W2):
    slot3, xs = _router(inp, Wg)
    slot = slot3.reshape(S)
    disp = _make_dispatch()(xs, slot)
    eout = _ffn(disp.reshape(NBLK, C, D // 2), W1, W2)
    out = _make_combine()(eout.reshape(NBLK * C, D), slot)
    return out
